# baseline (speedup 1.0000x reference)
import functools

import jax
import jax.numpy as jnp
from jax import lax
from jax.experimental import pallas as pl
from jax.experimental.pallas import tpu as pltpu
from jax.experimental.pallas import tpu_sc as plsc

N_NODES = 10000
N_EDGES = 320000
HIDDEN = 128

NC = 2
NS = 16
NW = NC * NS
EPW = N_EDGES // NW
K = 80
CC = 128
EPP = CC * K
PC = 32
NG = PC // 3
NP = 10240
RPT = NP // NS
HR = NP // HIDDEN
HRP = 128

_mesh = plsc.VectorSubcoreMesh(core_axis_name="c", subcore_axis_name="s")


def _fill_f32(ref, rows, cols, value):
    vecs = cols // 16

    def body(i, carry):
        r = i // vecs
        k = i % vecs
        ref[r, pl.ds(k * 16, 16)] = jnp.full((16,), value, jnp.float32)
        return carry

    lax.fori_loop(0, rows * vecs, body, 0)


@functools.partial(
    pl.kernel,
    out_type=jax.ShapeDtypeStruct((NC, HRP, HIDDEN), jnp.float32),
    mesh=_mesh,
    scratch_types=[
        pltpu.VMEM((EPW,), jnp.int32),
        pltpu.VMEM((HR, HIDDEN), jnp.float32),
        pltpu.VMEM((HR,), jnp.int32),
        pltpu.VMEM_SHARED((HRP, HIDDEN), jnp.float32),
    ],
    compiler_params=pltpu.CompilerParams(needs_layout_passes=False,
                                         disable_bounds_checks=True),
)
def _deg_kernel(dst_hbm, out_hbm, dst_v, hist_v, iota_v, acc_sh):
    c = lax.axis_index("c")
    s = lax.axis_index("s")
    w = c * NS + s

    pltpu.sync_copy(dst_hbm.at[w], dst_v)

    iota16 = lax.iota(jnp.int32, 16)
    zeros16 = jnp.zeros((16,), jnp.float32)
    ones16 = jnp.full((16,), 1.0, jnp.float32)

    def zbody(i, carry):
        flat = iota16 + i * 16
        plsc.store_scatter(hist_v, [flat >> 7, flat & 127], zeros16)
        return carry

    lax.fori_loop(0, NP // 16, zbody, 0)

    def ibody(q, carry):
        iota_v[pl.ds(q * 16, 16)] = iota16 + q * 16
        return carry

    lax.fori_loop(0, HR // 16, ibody, 0)

    pltpu.sync_copy(hist_v.at[pl.ds(0, HRP // NS)],
                    acc_sh.at[pl.ds(s * (HRP // NS), HRP // NS)])

    def body(i, carry):
        d = dst_v[pl.ds(i * 16, 16)]
        plsc.addupdate_scatter(hist_v, [d >> 7, d & 127], ones16)
        return carry

    lax.fori_loop(0, EPW // 16, body, 0)
    plsc.subcore_barrier()

    pltpu.sync_copy(hist_v, acc_sh.at[iota_v], add=True)
    plsc.subcore_barrier()

    pltpu.sync_copy(acc_sh.at[pl.ds(s * (HRP // NS), HRP // NS)],
                    out_hbm.at[c, pl.ds(s * (HRP // NS), HRP // NS)])


@functools.partial(
    pl.kernel,
    out_type=jax.ShapeDtypeStruct((NC, NP, HIDDEN), jnp.float32),
    mesh=_mesh,
    scratch_types=[
        pltpu.VMEM((PC, K), jnp.int32),
        pltpu.VMEM((PC, K), jnp.int32),
        pltpu.VMEM((K, HIDDEN), jnp.float32),
        pltpu.VMEM((K, HIDDEN), jnp.float32),
        pltpu.VMEM((K, HIDDEN), jnp.float32),
        pltpu.SemaphoreType.DMA,
        pltpu.SemaphoreType.DMA,
        pltpu.SemaphoreType.DMA,
        pltpu.SemaphoreType.DMA,
        pltpu.VMEM_SHARED((NP, HIDDEN), jnp.float32),
    ],
    compiler_params=pltpu.CompilerParams(disable_bounds_checks=True),
)
def _scatter_kernel(g_hbm, src_hbm, dst_hbm, out_hbm,
                    src_v, dst_v, r0, r1, r2, g0, g1, g2, ss, acc_sh):
    c = lax.axis_index("c")
    s = lax.axis_index("s")
    w = c * NS + s

    _fill_f32(r0, K, HIDDEN, 0.0)
    for q in range(RPT // K):
        pltpu.sync_copy(r0, acc_sh.at[pl.ds(s * RPT + q * K, K)])
    plsc.subcore_barrier()

    def _gather(j, rv, sem):
        return pltpu.async_copy(g_hbm.at[src_v.at[j]], rv, sem)

    def _gwait(j, rv, sem):
        pltpu.make_async_copy(g_hbm.at[src_v.at[j]], rv, sem).wait()

    def _scat(j, rv):
        pltpu.async_copy(rv, acc_sh.at[dst_v.at[j]], ss, add=True)

    def _sdrain(j, rv):
        pltpu.make_async_copy(rv, acc_sh.at[dst_v.at[j]], ss).wait()

    for p in range(CC // PC):
        pltpu.sync_copy(src_hbm.at[w, pl.ds(p * PC, PC)], src_v)
        pltpu.sync_copy(dst_hbm.at[w, pl.ds(p * PC, PC)], dst_v)

        def gbody(i, carry):
            j = i * 3
            _gather(j, r0, g0)
            _gather(j + 1, r1, g1)
            _gwait(j, r0, g0)
            _gather(j + 2, r2, g2)
            _gwait(j + 1, r1, g1)
            _gwait(j + 2, r2, g2)
            pltpu.sync_copy(r0, acc_sh.at[dst_v.at[j]], add=True)
            pltpu.sync_copy(r1, acc_sh.at[dst_v.at[j + 1]], add=True)
            pltpu.sync_copy(r2, acc_sh.at[dst_v.at[j + 2]], add=True)
            return carry

        lax.fori_loop(0, NG, gbody, 0)
        j = NG * 3
        _gather(j, r0, g0)
        _gather(j + 1, r1, g1)
        _gwait(j, r0, g0)
        _gwait(j + 1, r1, g1)
        pltpu.sync_copy(r0, acc_sh.at[dst_v.at[j]], add=True)
        pltpu.sync_copy(r1, acc_sh.at[dst_v.at[j + 1]], add=True)
    plsc.subcore_barrier()

    pltpu.sync_copy(acc_sh.at[pl.ds(s * RPT, RPT)],
                    out_hbm.at[c, pl.ds(s * RPT, RPT)])


def _dinv_t(deg_ref):
    degs = deg_ref[...]
    deg = degs[0] + degs[1] + 1.0
    return jnp.transpose(lax.rsqrt(deg))


def _tc1_body(x_ref, w_ref, deg_ref, g_ref):
    dt = _dinv_t(deg_ref)
    h = jnp.dot(x_ref[...], w_ref[...], preferred_element_type=jnp.float32)
    for q in range(dt.shape[1]):
        r = q * HIDDEN
        g_ref[r:r + HIDDEN, :] = h[r:r + HIDDEN, :] * dt[:, q:q + 1]


def _tc1(x, W, deg_p):
    blk = 1024
    grid = NP // blk
    db = blk // HIDDEN
    return pl.pallas_call(
        _tc1_body,
        grid=(grid,),
        in_specs=[
            pl.BlockSpec((blk, HIDDEN), lambda i: (i, 0)),
            pl.BlockSpec((HIDDEN, HIDDEN), lambda i: (0, 0)),
            pl.BlockSpec((NC, db, HIDDEN), lambda i: (0, i, 0)),
        ],
        out_specs=pl.BlockSpec((blk, HIDDEN), lambda i: (i, 0)),
        out_shape=jax.ShapeDtypeStruct((NP, HIDDEN), jnp.float32),
    )(x, W, deg_p)


def _tc2_body(acc_ref, g_ref, deg_ref, b_ref, o_ref):
    dt = _dinv_t(deg_ref)
    s = acc_ref[0] + acc_ref[1] + g_ref[...]
    bias = b_ref[...][None, :]
    for q in range(dt.shape[1]):
        r = q * HIDDEN
        o_ref[r:r + HIDDEN, :] = s[r:r + HIDDEN, :] * dt[:, q:q + 1] + bias


def _tc2(acc_p, g, deg_p, b):
    blk = 1024
    grid = NP // blk
    db = blk // HIDDEN
    return pl.pallas_call(
        _tc2_body,
        grid=(grid,),
        in_specs=[
            pl.BlockSpec((NC, blk, HIDDEN), lambda i: (0, i, 0)),
            pl.BlockSpec((blk, HIDDEN), lambda i: (i, 0)),
            pl.BlockSpec((NC, db, HIDDEN), lambda i: (0, i, 0)),
            pl.BlockSpec((HIDDEN,), lambda i: (0,)),
        ],
        out_specs=pl.BlockSpec((blk, HIDDEN), lambda i: (i, 0)),
        out_shape=jax.ShapeDtypeStruct((N_NODES, HIDDEN), jnp.float32),
    )(acc_p, g, deg_p, b)


def kernel(x, edge_index, W, b):
    ei = edge_index.astype(jnp.int32)
    src = jnp.concatenate(
        [ei[0].reshape(NW, EPW),
         jnp.zeros((NW, EPP - EPW), jnp.int32)], axis=1).reshape(NW, CC, K)
    dst = jnp.concatenate(
        [ei[1].reshape(NW, EPW),
         jnp.full((NW, EPP - EPW), N_NODES, jnp.int32)], axis=1
    ).reshape(NW, CC, K)
    deg_p = _deg_kernel(ei[1].reshape(NW, EPW))
    g = _tc1(x, W, deg_p)
    acc_p = _scatter_kernel(g, src, dst)
    return _tc2(acc_p, g, deg_p, b)


# speedup vs baseline: 1.9600x; 1.9600x over previous
import functools

import jax
import jax.numpy as jnp
from jax import lax
from jax.experimental import pallas as pl
from jax.experimental.pallas import tpu as pltpu
from jax.experimental.pallas import tpu_sc as plsc

N_NODES = 10000
N_EDGES = 320000
HIDDEN = 128

NC = 2
NS = 16
NW = NC * NS
EPW = N_EDGES // NW
K = 80
CC = EPW // K
EPP = CC * K
NP = 10240
RPT = NP // NS
HR = NP // HIDDEN
HRP = 128

_mesh = plsc.VectorSubcoreMesh(core_axis_name="c", subcore_axis_name="s")


def _fill_f32(ref, rows, cols, value):
    vecs = cols // 16

    def body(i, carry):
        r = i // vecs
        k = i % vecs
        ref[r, pl.ds(k * 16, 16)] = jnp.full((16,), value, jnp.float32)
        return carry

    lax.fori_loop(0, rows * vecs, body, 0)


@functools.partial(
    pl.kernel,
    out_type=jax.ShapeDtypeStruct((NC, HRP, HIDDEN), jnp.float32),
    mesh=_mesh,
    scratch_types=[
        pltpu.VMEM((EPW,), jnp.int32),
        pltpu.VMEM((HR, HIDDEN), jnp.float32),
        pltpu.VMEM((HR,), jnp.int32),
        pltpu.VMEM_SHARED((HRP, HIDDEN), jnp.float32),
    ],
    compiler_params=pltpu.CompilerParams(needs_layout_passes=False,
                                         disable_bounds_checks=True),
)
def _deg_kernel(dst_hbm, out_hbm, dst_v, hist_v, iota_v, acc_sh):
    c = lax.axis_index("c")
    s = lax.axis_index("s")
    w = c * NS + s

    pltpu.sync_copy(dst_hbm.at[w], dst_v)

    iota16 = lax.iota(jnp.int32, 16)
    zeros16 = jnp.zeros((16,), jnp.float32)
    ones16 = jnp.full((16,), 1.0, jnp.float32)

    def zbody(i, carry):
        flat = iota16 + i * 16
        plsc.store_scatter(hist_v, [flat >> 7, flat & 127], zeros16)
        return carry

    lax.fori_loop(0, NP // 16, zbody, 0)

    def ibody(q, carry):
        iota_v[pl.ds(q * 16, 16)] = iota16 + q * 16
        return carry

    lax.fori_loop(0, HR // 16, ibody, 0)

    pltpu.sync_copy(hist_v.at[pl.ds(0, HRP // NS)],
                    acc_sh.at[pl.ds(s * (HRP // NS), HRP // NS)])

    def body(i, carry):
        d = dst_v[pl.ds(i * 16, 16)]
        plsc.addupdate_scatter(hist_v, [d >> 7, d & 127], ones16)
        return carry

    lax.fori_loop(0, EPW // 16, body, 0)
    plsc.subcore_barrier()

    pltpu.sync_copy(hist_v, acc_sh.at[iota_v], add=True)
    plsc.subcore_barrier()

    pltpu.sync_copy(acc_sh.at[pl.ds(s * (HRP // NS), HRP // NS)],
                    out_hbm.at[c, pl.ds(s * (HRP // NS), HRP // NS)])


@functools.partial(
    pl.kernel,
    out_type=jax.ShapeDtypeStruct((NC, NP, HIDDEN), jnp.float32),
    mesh=_mesh,
    scratch_types=[
        pltpu.VMEM((CC, K), jnp.int32),
        pltpu.VMEM((CC, K), jnp.int32),
        pltpu.VMEM((K, HIDDEN), jnp.float32),
        pltpu.VMEM_SHARED((NP, HIDDEN), jnp.float32),
    ],
    compiler_params=pltpu.CompilerParams(disable_bounds_checks=True),
)
def _scatter_kernel(g_hbm, src_hbm, dst_hbm, out_hbm,
                    src_v, dst_v, rows_v, acc_sh):
    c = lax.axis_index("c")
    s = lax.axis_index("s")
    w = c * NS + s

    pltpu.sync_copy(src_hbm.at[w], src_v)
    pltpu.sync_copy(dst_hbm.at[w], dst_v)

    _fill_f32(rows_v, K, HIDDEN, 0.0)
    for q in range(RPT // K):
        pltpu.sync_copy(rows_v, acc_sh.at[pl.ds(s * RPT + q * K, K)])
    plsc.subcore_barrier()

    def body(j, carry):
        pltpu.sync_copy(g_hbm.at[src_v.at[j]], rows_v)
        pltpu.sync_copy(rows_v, acc_sh.at[dst_v.at[j]], add=True)
        return carry

    lax.fori_loop(0, CC, body, 0)
    plsc.subcore_barrier()

    pltpu.sync_copy(acc_sh.at[pl.ds(s * RPT, RPT)],
                    out_hbm.at[c, pl.ds(s * RPT, RPT)])


def _dinv_t(deg_ref):
    degs = deg_ref[...]
    deg = degs[0] + degs[1] + 1.0
    return jnp.transpose(lax.rsqrt(deg))


def _tc1_body(x_ref, w_ref, deg_ref, g_ref):
    dt = _dinv_t(deg_ref)
    h = jnp.dot(x_ref[...], w_ref[...], preferred_element_type=jnp.float32)
    for q in range(dt.shape[1]):
        r = q * HIDDEN
        g_ref[r:r + HIDDEN, :] = h[r:r + HIDDEN, :] * dt[:, q:q + 1]


def _tc1(x, W, deg_p):
    blk = 1024
    grid = NP // blk
    db = blk // HIDDEN
    return pl.pallas_call(
        _tc1_body,
        grid=(grid,),
        in_specs=[
            pl.BlockSpec((blk, HIDDEN), lambda i: (i, 0)),
            pl.BlockSpec((HIDDEN, HIDDEN), lambda i: (0, 0)),
            pl.BlockSpec((NC, db, HIDDEN), lambda i: (0, i, 0)),
        ],
        out_specs=pl.BlockSpec((blk, HIDDEN), lambda i: (i, 0)),
        out_shape=jax.ShapeDtypeStruct((NP, HIDDEN), jnp.float32),
    )(x, W, deg_p)


def _tc2_body(acc_ref, g_ref, deg_ref, b_ref, o_ref):
    dt = _dinv_t(deg_ref)
    s = acc_ref[0] + acc_ref[1] + g_ref[...]
    bias = b_ref[...][None, :]
    for q in range(dt.shape[1]):
        r = q * HIDDEN
        o_ref[r:r + HIDDEN, :] = s[r:r + HIDDEN, :] * dt[:, q:q + 1] + bias


def _tc2(acc_p, g, deg_p, b):
    blk = 1024
    grid = NP // blk
    db = blk // HIDDEN
    return pl.pallas_call(
        _tc2_body,
        grid=(grid,),
        in_specs=[
            pl.BlockSpec((NC, blk, HIDDEN), lambda i: (0, i, 0)),
            pl.BlockSpec((blk, HIDDEN), lambda i: (i, 0)),
            pl.BlockSpec((NC, db, HIDDEN), lambda i: (0, i, 0)),
            pl.BlockSpec((HIDDEN,), lambda i: (0,)),
        ],
        out_specs=pl.BlockSpec((blk, HIDDEN), lambda i: (i, 0)),
        out_shape=jax.ShapeDtypeStruct((N_NODES, HIDDEN), jnp.float32),
    )(acc_p, g, deg_p, b)


def kernel(x, edge_index, W, b):
    ei = edge_index.astype(jnp.int32)
    src = jnp.concatenate(
        [ei[0].reshape(NW, EPW),
         jnp.zeros((NW, EPP - EPW), jnp.int32)], axis=1).reshape(NW, CC, K)
    dst = jnp.concatenate(
        [ei[1].reshape(NW, EPW),
         jnp.full((NW, EPP - EPW), N_NODES, jnp.int32)], axis=1
    ).reshape(NW, CC, K)
    deg_p = _deg_kernel(ei[1].reshape(NW, EPW))
    g = _tc1(x, W, deg_p)
    acc_p = _scatter_kernel(g, src, dst)
    return _tc2(acc_p, g, deg_p, b)
